# baseline (speedup 1.0000x reference)
import functools

import jax
import jax.numpy as jnp
from jax import lax
from jax.experimental import pallas as pl
from jax.experimental.pallas import tpu as pltpu
from jax.experimental.pallas import tpu_sc as plsc

N = 10000
E = 320000
D = 128
NMASK = 1000

NC = 2
NS = 16
NW = NC * NS
CH = 128
NCHK = 2560
EP = NCHK * CH
CPW = NCHK // NW
EPW = CPW * CH
RPT = N // NS
ZCH = 125

f32 = jnp.float32
i32 = jnp.int32

_mesh = plsc.VectorSubcoreMesh(core_axis_name="c", subcore_axis_name="s")


@functools.partial(
    pl.kernel,
    out_type=jax.ShapeDtypeStruct((NC * N,), f32),
    mesh=_mesh,
    scratch_types=[
        pltpu.VMEM((CPW, CH), i32),
        pltpu.VMEM((EPW,), f32),
        pltpu.VMEM((1024,), f32),
        pltpu.VMEM_SHARED((N,), f32),
        pltpu.SemaphoreType.DMA,
        pltpu.SemaphoreType.DMA,
    ],
)
def _deg_kernel(dst2d_hbm, ew_hbm, out_hbm, idx_d, ewb, zb, sdeg, isem, ssem):
    c = lax.axis_index("c")
    s = lax.axis_index("s")
    wid = s * NC + c

    pltpu.async_copy(dst2d_hbm.at[pl.ds(wid * CPW, CPW)], idx_d, isem)
    pltpu.async_copy(ew_hbm.at[pl.ds(wid * EPW, EPW)], ewb, isem)

    def zb_body(i, carry):
        zb[pl.ds(i * 16, 16)] = jnp.zeros((16,), f32)
        return carry

    lax.fori_loop(0, 64, zb_body, 0)

    @pl.when(s < 10)
    def _():
        pltpu.sync_copy(zb.at[pl.ds(0, 1000)], sdeg.at[pl.ds(s * 1000, 1000)])

    plsc.subcore_barrier()
    pltpu.make_async_copy(dst2d_hbm.at[pl.ds(0, CPW)], idx_d, isem).wait()
    pltpu.make_async_copy(ew_hbm.at[pl.ds(0, EPW)], ewb, isem).wait()

    LAG = 6

    def chunk(i, carry):
        pltpu.async_copy(ewb.at[pl.ds(i * CH, CH)], sdeg.at[idx_d.at[i]],
                         ssem, add=True)

        @pl.when(i >= LAG)
        def _():
            pltpu.make_async_copy(ewb.at[pl.ds(0, CH)],
                                  sdeg.at[idx_d.at[0]], ssem).wait()

        return carry

    lax.fori_loop(0, CPW, chunk, 0)
    for _ in range(LAG):
        pltpu.make_async_copy(ewb.at[pl.ds(0, CH)],
                              sdeg.at[idx_d.at[0]], ssem).wait()

    plsc.subcore_barrier()

    @pl.when(s < 10)
    def _():
        pltpu.sync_copy(sdeg.at[pl.ds(s * 1000, 1000)], zb.at[pl.ds(0, 1000)])
        pltpu.sync_copy(zb.at[pl.ds(0, 1000)],
                        out_hbm.at[pl.ds(c * N + s * 1000, 1000)])


@functools.partial(
    pl.kernel,
    out_type=jax.ShapeDtypeStruct((NC, N, D), f32),
    mesh=_mesh,
    scratch_types=[
        pltpu.VMEM((CH,), i32),
        pltpu.VMEM((CH,), i32),
        pltpu.VMEM((CH,), f32),
        pltpu.VMEM((CH, D), f32),
        pltpu.VMEM_SHARED((N, D), f32),
        pltpu.SemaphoreType.DMA,
    ],
)
def _agg_kernel(src_hbm, dst_hbm, ew_hbm, g_hbm, out_hbm,
                idx_s, idx_d, ewb, rows, acc, sem):
    c = lax.axis_index("c")
    s = lax.axis_index("s")
    wid = s * NC + c

    def zrow(i, carry):
        for k8 in range(8):
            rows[i, pl.ds(k8 * 16, 16)] = jnp.zeros((16,), f32)
        return carry

    lax.fori_loop(0, CH, zrow, 0)

    rbase = s * RPT
    for k in range(4):
        pltpu.sync_copy(rows.at[pl.ds(0, CH)],
                        acc.at[pl.ds(rbase + k * CH, CH)])
    pltpu.sync_copy(rows.at[pl.ds(0, 113)],
                    acc.at[pl.ds(rbase + 4 * CH, 113)])
    plsc.subcore_barrier()

    base0 = wid * EPW

    def scale_full(jv, carry):
        ew16 = ewb[pl.ds(jv * 16, 16)]
        for lane in range(16):
            sc = ew16[lane]
            r = jv * 16 + lane
            for k8 in range(8):
                sl = pl.ds(k8 * 16, 16)
                rows[r, sl] = rows[r, sl] * sc
        return carry

    def chunk(i, carry):
        base = base0 + i * CH
        pltpu.sync_copy(src_hbm.at[pl.ds(base, CH)], idx_s)
        pltpu.sync_copy(dst_hbm.at[pl.ds(base, CH)], idx_d)
        pltpu.sync_copy(ew_hbm.at[pl.ds(base, CH)], ewb)
        pltpu.async_copy(g_hbm.at[idx_s], rows, sem).wait()
        lax.fori_loop(0, CH // 16, scale_full, 0)
        pltpu.sync_copy(rows, acc.at[idx_d], add=True)
        return carry

    lax.fori_loop(0, CPW, chunk, 0)

    plsc.subcore_barrier()
    obase = s * 624
    for k in range(4):
        pltpu.sync_copy(acc.at[pl.ds(obase + k * CH, CH)],
                        rows.at[pl.ds(0, CH)])
        pltpu.sync_copy(rows.at[pl.ds(0, CH)],
                        out_hbm.at[c, pl.ds(obase + k * CH, CH)])
    pltpu.sync_copy(acc.at[pl.ds(obase + 4 * CH, 112)],
                    rows.at[pl.ds(0, 112)])
    pltpu.sync_copy(rows.at[pl.ds(0, 112)],
                    out_hbm.at[c, pl.ds(obase + 4 * CH, 112)])

    @pl.when(s == 0)
    def _():
        pltpu.sync_copy(acc.at[pl.ds(9984, 16)], rows.at[pl.ds(0, 16)])
        pltpu.sync_copy(rows.at[pl.ds(0, 16)],
                        out_hbm.at[c, pl.ds(9984, 16)])


MW = 25
MR = 40


@functools.partial(
    pl.kernel,
    out_type=(jax.ShapeDtypeStruct((NMASK, D), f32),
              jax.ShapeDtypeStruct((NMASK,), i32)),
    mesh=_mesh,
    scratch_types=[
        pltpu.VMEM((MR,), i32),
        pltpu.VMEM((MR, D), f32),
        pltpu.VMEM((MR,), i32),
        pltpu.SemaphoreType.DMA,
    ],
)
def _mask_kernel(outf_hbm, mask_hbm, y_hbm, om_hbm, ym_hbm,
                 midx, rowb, yb, sem):
    c = lax.axis_index("c")
    s = lax.axis_index("s")
    wid = s * NC + c

    @pl.when(wid < MW)
    def _():
        base = wid * MR
        pltpu.sync_copy(mask_hbm.at[pl.ds(base, MR)], midx)
        pltpu.async_copy(outf_hbm.at[midx], rowb, sem).wait()
        pltpu.sync_copy(rowb, om_hbm.at[pl.ds(base, MR)])
        pltpu.async_copy(y_hbm.at[midx], yb, sem).wait()
        pltpu.sync_copy(yb, ym_hbm.at[pl.ds(base, MR)])


BR = 2000


def _lin1_body(x_ref, w_ref, dp_ref, g_ref, dis_ref):
    deg = dp_ref[0] + dp_ref[1] + 1.0
    dis = jnp.where(deg > 0, lax.rsqrt(jnp.maximum(deg, 1e-12)), 0.0)
    h = jnp.dot(x_ref[...], w_ref[...], preferred_element_type=f32)
    g_ref[...] = h * dis
    dis_ref[...] = dis


def _lin1(x, W1, dp3):
    return pl.pallas_call(
        _lin1_body,
        grid=(N // BR,),
        in_specs=[
            pl.BlockSpec((BR, D), lambda i: (i, 0)),
            pl.BlockSpec((D, D), lambda i: (0, 0)),
            pl.BlockSpec((2, BR, 1), lambda i: (0, i, 0)),
        ],
        out_specs=[
            pl.BlockSpec((BR, D), lambda i: (i, 0)),
            pl.BlockSpec((BR, 1), lambda i: (i, 0)),
        ],
        out_shape=[
            jax.ShapeDtypeStruct((N, D), f32),
            jax.ShapeDtypeStruct((N, 1), f32),
        ],
    )(x, W1, dp3)


def _lin2_body(p_ref, g1_ref, dis_ref, b1_ref, w2_ref, g2_ref):
    t = dis_ref[...] * (p_ref[0] + p_ref[1] + g1_ref[...]) + b1_ref[...]
    z = jnp.maximum(t, 0.0)
    g2_ref[...] = jnp.dot(z, w2_ref[...],
                          preferred_element_type=f32) * dis_ref[...]


def _lin2(p, g1, dis, b1r, W2):
    return pl.pallas_call(
        _lin2_body,
        grid=(N // BR,),
        in_specs=[
            pl.BlockSpec((2, BR, D), lambda i: (0, i, 0)),
            pl.BlockSpec((BR, D), lambda i: (i, 0)),
            pl.BlockSpec((BR, 1), lambda i: (i, 0)),
            pl.BlockSpec((1, D), lambda i: (0, 0)),
            pl.BlockSpec((D, D), lambda i: (0, 0)),
        ],
        out_specs=pl.BlockSpec((BR, D), lambda i: (i, 0)),
        out_shape=jax.ShapeDtypeStruct((N, D), f32),
    )(p, g1, dis, b1r, W2)


def _combine_body(q_ref, g2_ref, dis_ref, b2_ref, o_ref):
    o_ref[...] = dis_ref[...] * (q_ref[0] + q_ref[1] + g2_ref[...]) \
        + b2_ref[...]


def _combine(q, g2, dis, b2r):
    return pl.pallas_call(
        _combine_body,
        grid=(N // BR,),
        in_specs=[
            pl.BlockSpec((2, BR, D), lambda i: (0, i, 0)),
            pl.BlockSpec((BR, D), lambda i: (i, 0)),
            pl.BlockSpec((BR, 1), lambda i: (i, 0)),
            pl.BlockSpec((1, D), lambda i: (0, 0)),
        ],
        out_specs=pl.BlockSpec((BR, D), lambda i: (i, 0)),
        out_shape=jax.ShapeDtypeStruct((N, D), f32),
    )(q, g2, dis, b2r)


def kernel(x, edge_index, edge_weight, mask_idx, y, W1, b1, W2, b2):
    pad = EP - E
    zpad_i = jnp.zeros((pad,), i32)
    src_idx = jnp.concatenate([edge_index[0], zpad_i])
    dst_idx = jnp.concatenate([edge_index[1], zpad_i])
    ew_p = jnp.concatenate([edge_weight, jnp.zeros((pad,), f32)])
    dst2d = dst_idx.reshape(NCHK, CH)

    dp = _deg_kernel(dst2d, ew_p)
    dp3 = dp.reshape(2, N, 1)
    g1, dis = _lin1(x, W1, dp3)
    p = _agg_kernel(src_idx, dst_idx, ew_p, g1)
    g2 = _lin2(p, g1, dis, b1.reshape(1, D), W2)
    q = _agg_kernel(src_idx, dst_idx, ew_p, g2)
    outf = _combine(q, g2, dis, b2.reshape(1, D))
    out_m, y_m = _mask_kernel(outf, mask_idx, y)
    return (out_m, y_m)


# speedup vs baseline: 4.1350x; 4.1350x over previous
import functools

import jax
import jax.numpy as jnp
from jax import lax
from jax.experimental import pallas as pl
from jax.experimental.pallas import tpu as pltpu
from jax.experimental.pallas import tpu_sc as plsc

N = 10000
E = 320000
D = 128
NMASK = 1000

NC = 2
NS = 16
NW = NC * NS
CH = 64
NCHK = 5120
EP = NCHK * CH
CPW = NCHK // NW
EPW = CPW * CH
RPT = N // NS
ZCH = 125

f32 = jnp.float32
i32 = jnp.int32

_mesh = plsc.VectorSubcoreMesh(core_axis_name="c", subcore_axis_name="s")


@functools.partial(
    pl.kernel,
    out_type=jax.ShapeDtypeStruct((NC * N,), f32),
    mesh=_mesh,
    scratch_types=[
        pltpu.VMEM((CPW, CH), i32),
        pltpu.VMEM((EPW,), f32),
        pltpu.VMEM((1024,), f32),
        pltpu.VMEM_SHARED((N,), f32),
        pltpu.SemaphoreType.DMA,
        pltpu.SemaphoreType.DMA,
    ],
)
def _deg_kernel(dst2d_hbm, ew_hbm, out_hbm, idx_d, ewb, zb, sdeg, isem, ssem):
    c = lax.axis_index("c")
    s = lax.axis_index("s")
    wid = s * NC + c

    pltpu.async_copy(dst2d_hbm.at[pl.ds(wid * CPW, CPW)], idx_d, isem)
    pltpu.async_copy(ew_hbm.at[pl.ds(wid * EPW, EPW)], ewb, isem)

    def zb_body(i, carry):
        zb[pl.ds(i * 16, 16)] = jnp.zeros((16,), f32)
        return carry

    lax.fori_loop(0, 64, zb_body, 0)

    @pl.when(s < 10)
    def _():
        pltpu.sync_copy(zb.at[pl.ds(0, 1000)], sdeg.at[pl.ds(s * 1000, 1000)])

    plsc.subcore_barrier()
    pltpu.make_async_copy(dst2d_hbm.at[pl.ds(0, CPW)], idx_d, isem).wait()
    pltpu.make_async_copy(ew_hbm.at[pl.ds(0, EPW)], ewb, isem).wait()

    LAG = 6

    def chunk(i, carry):
        pltpu.async_copy(ewb.at[pl.ds(i * CH, CH)], sdeg.at[idx_d.at[i]],
                         ssem, add=True)

        @pl.when(i >= LAG)
        def _():
            pltpu.make_async_copy(ewb.at[pl.ds(0, CH)],
                                  sdeg.at[idx_d.at[0]], ssem).wait()

        return carry

    lax.fori_loop(0, CPW, chunk, 0)
    for _ in range(LAG):
        pltpu.make_async_copy(ewb.at[pl.ds(0, CH)],
                              sdeg.at[idx_d.at[0]], ssem).wait()

    plsc.subcore_barrier()

    @pl.when(s < 10)
    def _():
        pltpu.sync_copy(sdeg.at[pl.ds(s * 1000, 1000)], zb.at[pl.ds(0, 1000)])
        pltpu.sync_copy(zb.at[pl.ds(0, 1000)],
                        out_hbm.at[pl.ds(c * N + s * 1000, 1000)])


RING = 8
NB = 4


@functools.partial(
    pl.kernel,
    out_type=jax.ShapeDtypeStruct((NC, N, D), f32),
    mesh=_mesh,
    scratch_types=[
        pltpu.VMEM((RING, CH), i32),
        pltpu.VMEM((24, CH), i32),
        pltpu.VMEM((RING, CH), f32),
        pltpu.VMEM((NB, CH, D), f32),
        pltpu.VMEM_SHARED((N, D), f32),
        pltpu.SemaphoreType.DMA,
        pltpu.SemaphoreType.DMA,
        pltpu.SemaphoreType.DMA,
        pltpu.SemaphoreType.DMA,
        pltpu.SemaphoreType.DMA,
        pltpu.SemaphoreType.DMA,
        pltpu.SemaphoreType.DMA,
        pltpu.SemaphoreType.DMA,
        pltpu.SemaphoreType.DMA,
        pltpu.SemaphoreType.DMA,
    ],
)
def _agg_kernel(src_hbm, dst2d_hbm, ew_hbm, g_hbm, out_hbm,
                idx_s, idx_d, ewb, rows, acc, psem, dsem,
                g0, g1, g2, g3, s0, s1, s2, s3):
    gsems = (g0, g1, g2, g3)
    ssems = (s0, s1, s2, s3)
    c = lax.axis_index("c")
    s = lax.axis_index("s")
    wid = s * NC + c
    ebase = wid * EPW
    cbase = wid * CPW

    def issue_src(j):
        r = lax.rem(j, RING)
        pltpu.async_copy(src_hbm.at[pl.ds(ebase + j * CH, CH)],
                         idx_s.at[r], psem)
        pltpu.async_copy(ew_hbm.at[pl.ds(ebase + j * CH, CH)],
                         ewb.at[r], psem)

    def wait_src():
        for _ in range(2):
            pltpu.make_async_copy(src_hbm.at[pl.ds(0, CH)],
                                  idx_s.at[0], psem).wait()

    def issue_dstblk(k):
        half = lax.rem(k, 3) * 8
        pltpu.async_copy(dst2d_hbm.at[pl.ds(cbase + k * 8, 8)],
                         idx_d.at[pl.ds(half, 8)], dsem)

    def wait_dstblk():
        pltpu.make_async_copy(dst2d_hbm.at[pl.ds(0, 8)],
                              idx_d.at[pl.ds(0, 8)], dsem).wait()

    def issue_gather(j, b):
        pltpu.async_copy(g_hbm.at[idx_s.at[lax.rem(j, RING)]],
                         rows.at[b], gsems[b])

    def wait_gather(b):
        pltpu.make_async_copy(g_hbm.at[idx_s.at[0]],
                              rows.at[b], gsems[b]).wait()

    def issue_scatter(j, b):
        drow = lax.rem(j // 8, 3) * 8 + lax.rem(j, 8)
        pltpu.async_copy(rows.at[b], acc.at[idx_d.at[drow]],
                         ssems[b], add=True)

    def wait_scatter(b):
        pltpu.make_async_copy(rows.at[b], acc.at[idx_d.at[0]],
                              ssems[b]).wait()

    def scale(j, b):
        rr = lax.rem(j, RING)

        def grp(jv, carry):
            ew16 = ewb[rr, pl.ds(jv * 16, 16)]
            for lane in range(16):
                sc = ew16[lane]
                r = jv * 16 + lane
                for k8 in range(8):
                    sl = pl.ds(k8 * 16, 16)
                    rows[b, r, sl] = rows[b, r, sl] * sc
            return carry

        lax.fori_loop(0, CH // 16, grp, 0)

    for j in range(5):
        issue_src(j)
    issue_dstblk(0)
    issue_dstblk(1)

    def zrow(i, carry):
        for k8 in range(8):
            rows[0, i, pl.ds(k8 * 16, 16)] = jnp.zeros((16,), f32)
        return carry

    lax.fori_loop(0, CH, zrow, 0)

    rbase = s * RPT
    for k in range(9):
        pltpu.sync_copy(rows.at[0, pl.ds(0, CH)],
                        acc.at[pl.ds(rbase + k * CH, CH)])
    pltpu.sync_copy(rows.at[0, pl.ds(0, 49)],
                    acc.at[pl.ds(rbase + 9 * CH, 49)])
    plsc.subcore_barrier()

    for j in range(3):
        wait_src()
        issue_gather(j, j)

    def step(i, b, first=False):
        if not first:
            wait_scatter((b + 3) % NB)

        @pl.when(lax.rem(i, 8) == 0)
        def _():
            wait_dstblk()

            @pl.when(i + 16 < CPW)
            def _():
                issue_dstblk((i + 16) // 8)

        wait_gather(b)

        @pl.when(i + 3 < CPW)
        def _():
            wait_src()
            issue_gather(i + 3, (b + 3) % NB)

        @pl.when(i + 5 < CPW)
        def _():
            issue_src(i + 5)

        scale(i, b)
        issue_scatter(i, b)

    for i in range(NB):
        step(i, i, first=(i == 0))

    def body(g, carry):
        i0 = NB * (g + 1)
        for t in range(NB):
            step(i0 + t, t)
        return carry

    lax.fori_loop(0, CPW // NB - 1, body, 0)
    wait_scatter(3)

    plsc.subcore_barrier()
    obase = s * 624
    for k in range(9):
        pltpu.sync_copy(acc.at[pl.ds(obase + k * CH, CH)],
                        rows.at[0, pl.ds(0, CH)])
        pltpu.sync_copy(rows.at[0, pl.ds(0, CH)],
                        out_hbm.at[c, pl.ds(obase + k * CH, CH)])
    pltpu.sync_copy(acc.at[pl.ds(obase + 9 * CH, 48)],
                    rows.at[0, pl.ds(0, 48)])
    pltpu.sync_copy(rows.at[0, pl.ds(0, 48)],
                    out_hbm.at[c, pl.ds(obase + 9 * CH, 48)])

    @pl.when(s == 0)
    def _():
        pltpu.sync_copy(acc.at[pl.ds(9984, 16)], rows.at[0, pl.ds(0, 16)])
        pltpu.sync_copy(rows.at[0, pl.ds(0, 16)],
                        out_hbm.at[c, pl.ds(9984, 16)])


MW = 25
MR = 40


@functools.partial(
    pl.kernel,
    out_type=(jax.ShapeDtypeStruct((NMASK, D), f32),
              jax.ShapeDtypeStruct((NMASK,), i32)),
    mesh=_mesh,
    scratch_types=[
        pltpu.VMEM((MR,), i32),
        pltpu.VMEM((MR, D), f32),
        pltpu.VMEM((MR,), i32),
        pltpu.SemaphoreType.DMA,
    ],
)
def _mask_kernel(outf_hbm, mask_hbm, y_hbm, om_hbm, ym_hbm,
                 midx, rowb, yb, sem):
    c = lax.axis_index("c")
    s = lax.axis_index("s")
    wid = s * NC + c

    @pl.when(wid < MW)
    def _():
        base = wid * MR
        pltpu.sync_copy(mask_hbm.at[pl.ds(base, MR)], midx)
        pltpu.async_copy(outf_hbm.at[midx], rowb, sem).wait()
        pltpu.sync_copy(rowb, om_hbm.at[pl.ds(base, MR)])
        pltpu.async_copy(y_hbm.at[midx], yb, sem).wait()
        pltpu.sync_copy(yb, ym_hbm.at[pl.ds(base, MR)])


BR = 2000


def _lin1_body(x_ref, w_ref, dp_ref, g_ref, dis_ref):
    deg = dp_ref[0] + dp_ref[1] + 1.0
    dis = jnp.where(deg > 0, lax.rsqrt(jnp.maximum(deg, 1e-12)), 0.0)
    h = jnp.dot(x_ref[...], w_ref[...], preferred_element_type=f32)
    g_ref[...] = h * dis
    dis_ref[...] = dis


def _lin1(x, W1, dp3):
    return pl.pallas_call(
        _lin1_body,
        grid=(N // BR,),
        in_specs=[
            pl.BlockSpec((BR, D), lambda i: (i, 0)),
            pl.BlockSpec((D, D), lambda i: (0, 0)),
            pl.BlockSpec((2, BR, 1), lambda i: (0, i, 0)),
        ],
        out_specs=[
            pl.BlockSpec((BR, D), lambda i: (i, 0)),
            pl.BlockSpec((BR, 1), lambda i: (i, 0)),
        ],
        out_shape=[
            jax.ShapeDtypeStruct((N, D), f32),
            jax.ShapeDtypeStruct((N, 1), f32),
        ],
    )(x, W1, dp3)


def _lin2_body(p_ref, g1_ref, dis_ref, b1_ref, w2_ref, g2_ref):
    t = dis_ref[...] * (p_ref[0] + p_ref[1] + g1_ref[...]) + b1_ref[...]
    z = jnp.maximum(t, 0.0)
    g2_ref[...] = jnp.dot(z, w2_ref[...],
                          preferred_element_type=f32) * dis_ref[...]


def _lin2(p, g1, dis, b1r, W2):
    return pl.pallas_call(
        _lin2_body,
        grid=(N // BR,),
        in_specs=[
            pl.BlockSpec((2, BR, D), lambda i: (0, i, 0)),
            pl.BlockSpec((BR, D), lambda i: (i, 0)),
            pl.BlockSpec((BR, 1), lambda i: (i, 0)),
            pl.BlockSpec((1, D), lambda i: (0, 0)),
            pl.BlockSpec((D, D), lambda i: (0, 0)),
        ],
        out_specs=pl.BlockSpec((BR, D), lambda i: (i, 0)),
        out_shape=jax.ShapeDtypeStruct((N, D), f32),
    )(p, g1, dis, b1r, W2)


def _combine_body(q_ref, g2_ref, dis_ref, b2_ref, o_ref):
    o_ref[...] = dis_ref[...] * (q_ref[0] + q_ref[1] + g2_ref[...]) \
        + b2_ref[...]


def _combine(q, g2, dis, b2r):
    return pl.pallas_call(
        _combine_body,
        grid=(N // BR,),
        in_specs=[
            pl.BlockSpec((2, BR, D), lambda i: (0, i, 0)),
            pl.BlockSpec((BR, D), lambda i: (i, 0)),
            pl.BlockSpec((BR, 1), lambda i: (i, 0)),
            pl.BlockSpec((1, D), lambda i: (0, 0)),
        ],
        out_specs=pl.BlockSpec((BR, D), lambda i: (i, 0)),
        out_shape=jax.ShapeDtypeStruct((N, D), f32),
    )(q, g2, dis, b2r)


def kernel(x, edge_index, edge_weight, mask_idx, y, W1, b1, W2, b2):
    pad = EP - E
    spread = jnp.arange(pad, dtype=i32) % N
    src_idx = jnp.concatenate([edge_index[0], spread])
    dst_idx = jnp.concatenate([edge_index[1], spread])
    ew_p = jnp.concatenate([edge_weight, jnp.zeros((pad,), f32)])
    dst2d = dst_idx.reshape(NCHK, CH)

    dp = _deg_kernel(dst2d, ew_p)
    dp3 = dp.reshape(2, N, 1)
    g1, dis = _lin1(x, W1, dp3)
    p = _agg_kernel(src_idx, dst2d, ew_p, g1)
    g2 = _lin2(p, g1, dis, b1.reshape(1, D), W2)
    q = _agg_kernel(src_idx, dst2d, ew_p, g2)
    outf = _combine(q, g2, dis, b2.reshape(1, D))
    out_m, y_m = _mask_kernel(outf, mask_idx, y)
    return (out_m, y_m)
